# baseline (speedup 1.0000x reference)
import jax
import jax.numpy as jnp
from jax import lax
from jax.experimental import pallas as pl
from jax.experimental.pallas import tpu as pltpu
from jax.experimental.pallas import tpu_sc as plsc

B = 16384
D = 1024
LANES = 16
NC = 2
NS = 16
NW = NC * NS
SC_ROWS = 10240
TC_ROWS = B - SC_ROWS
B_PER_W = SC_ROWS // NW
N_GRP = B_PER_W // LANES
DRAIN_ROWS = 16
BLK = 256


def _sc_body(idx_hbm, table_hbm, out_hbm, idx_v, table_v, drain_v, sem):
    sid = lax.axis_index("s")
    wid = sid * NC + lax.axis_index("c")
    base = wid * B_PER_W

    pltpu.sync_copy(table_hbm, table_v)
    pltpu.sync_copy(idx_hbm.at[pl.ds(base, B_PER_W)], idx_v)

    def grp_body(g, carry):
        row = base + g * LANES
        tvec = idx_v[pl.ds(g * LANES, LANES)]
        for j in range(LANES):
            pltpu.async_copy(table_v.at[tvec[j]], out_hbm.at[row + j], sem)
        return carry

    lax.fori_loop(0, N_GRP, grp_body, 0)

    def drain_body(i, carry):
        pltpu.make_async_copy(out_hbm.at[pl.ds(base, DRAIN_ROWS)], drain_v, sem).wait()
        return carry

    lax.fori_loop(0, B_PER_W // DRAIN_ROWS, drain_body, 0)


_sc_lookup = pl.kernel(
    _sc_body,
    out_type=jax.ShapeDtypeStruct((SC_ROWS, D), jnp.float32),
    mesh=plsc.VectorSubcoreMesh(core_axis_name="c", subcore_axis_name="s"),
    scratch_types=[
        pltpu.VMEM((B_PER_W,), jnp.int32),
        pltpu.VMEM((2, D), jnp.float32),
        pltpu.VMEM((DRAIN_ROWS, D), jnp.float32),
        pltpu.SemaphoreType.DMA,
    ],
)


def _tc_body(idx_ref, tab_ref, o_ref):
    idxb = idx_ref[0]
    w0 = tab_ref[pl.ds(0, 1), :]
    w1 = tab_ref[pl.ds(1, 1), :]
    o_ref[...] = jnp.where(idxb == 0, w0, w1)


def kernel(domain_idx, embed_weight):
    idx = domain_idx.astype(jnp.int32)
    sc_out = _sc_lookup(idx[:SC_ROWS], embed_weight)
    idx3 = idx[SC_ROWS:].reshape(TC_ROWS // BLK, BLK, 1)
    tc_out = pl.pallas_call(
        _tc_body,
        out_shape=jax.ShapeDtypeStruct((TC_ROWS, D), jnp.float32),
        grid=(TC_ROWS // BLK,),
        in_specs=[
            pl.BlockSpec((1, BLK, 1), lambda i: (i, 0, 0)),
            pl.BlockSpec((2, D), lambda i: (0, 0)),
        ],
        out_specs=pl.BlockSpec((BLK, D), lambda i: (i, 0)),
    )(idx3, embed_weight)
    return jnp.concatenate([sc_out, tc_out], axis=0)


# speedup vs baseline: 2.3421x; 2.3421x over previous
import jax
import jax.numpy as jnp
from jax import lax
from jax.experimental import pallas as pl
from jax.experimental.pallas import tpu as pltpu
from jax.experimental.pallas import tpu_sc as plsc

B = 16384
D = 1024
LANES = 16
NC = 2
NS = 16
NW = NC * NS
B_PER_W = B // NW
N_GRP = B_PER_W // LANES
DRAIN_ROWS = 32


def _body(idx_hbm, table_hbm, out_hbm, idx_v, table_v, drain_v, sem, sem_in):
    sid = lax.axis_index("s")
    wid = sid * NC + lax.axis_index("c")
    base = wid * B_PER_W

    h_tab = pltpu.async_copy(table_hbm, table_v, sem_in)
    h_idx = pltpu.async_copy(idx_hbm.at[pl.ds(base, B_PER_W)], idx_v, sem_in)
    h_tab.wait()
    h_idx.wait()

    def grp_body(g, carry):
        row = base + g * LANES
        tvec = idx_v[pl.ds(g * LANES, LANES)]
        for j in range(LANES):
            pltpu.async_copy(table_v.at[tvec[j]], out_hbm.at[row + j], sem)
        return carry

    lax.fori_loop(0, N_GRP, grp_body, 0)

    def drain_body(i, carry):
        pltpu.make_async_copy(out_hbm.at[pl.ds(base, DRAIN_ROWS)], drain_v, sem).wait()
        return carry

    lax.fori_loop(0, B_PER_W // DRAIN_ROWS, drain_body, 0)


_sc_lookup = pl.kernel(
    _body,
    out_type=jax.ShapeDtypeStruct((B, D), jnp.float32),
    mesh=plsc.VectorSubcoreMesh(core_axis_name="c", subcore_axis_name="s"),
    scratch_types=[
        pltpu.VMEM((B_PER_W,), jnp.int32),
        pltpu.VMEM((2, D), jnp.float32),
        pltpu.VMEM((DRAIN_ROWS, D), jnp.float32),
        pltpu.SemaphoreType.DMA,
        pltpu.SemaphoreType.DMA,
    ],
)


def kernel(domain_idx, embed_weight):
    return _sc_lookup(domain_idx.astype(jnp.int32), embed_weight)
